# Initial kernel scaffold; baseline (speedup 1.0000x reference)
import jax
import jax.numpy as jnp
from jax.experimental import pallas as pl


def kernel(x, edge_index, edge_weight, W1, W2):
    raise NotImplementedError("write your pallas kernel here")


# speedup vs baseline: 6.5320x; 6.5320x over previous
import functools

import jax
import jax.numpy as jnp
from jax import lax
from jax.experimental import pallas as pl
from jax.experimental.pallas import tpu as pltpu
from jax.experimental.pallas import tpu_sc as plsc

N_NODES = 10000
N_PAD = 10240
N_EDGES = 320000
D_FEAT = 128
D_HALF = 64
D_OUT = 16

NTILE = 16
NCH = 250
CH = 80
RPT = N_PAD // NTILE

_SC_PARAMS = pltpu.CompilerParams(use_tc_tiling_on_sc=False)


def _mm1_body(x_ref, w_ref, o0_ref, o1_ref):
    res = jnp.dot(x_ref[...], w_ref[...], preferred_element_type=jnp.float32)
    o0_ref[...] = res[:, :D_HALF]
    o1_ref[...] = res[:, D_HALF:]


def _matmul1(x, w1):
    bm = 1000
    return pl.pallas_call(
        _mm1_body,
        grid=(N_NODES // bm,),
        in_specs=[
            pl.BlockSpec((bm, D_FEAT), lambda i: (i, 0)),
            pl.BlockSpec((D_FEAT, D_FEAT), lambda i: (0, 0)),
        ],
        out_specs=[
            pl.BlockSpec((bm, D_HALF), lambda i: (i, 0)),
            pl.BlockSpec((bm, D_HALF), lambda i: (i, 0)),
        ],
        out_shape=[
            jax.ShapeDtypeStruct((N_NODES, D_HALF), jnp.float32),
            jax.ShapeDtypeStruct((N_NODES, D_HALF), jnp.float32),
        ],
    )(x, w1)


def _mm2_body(h0_ref, h1_ref, wa_ref, wb_ref, o_ref):
    a = jnp.maximum(h0_ref[...], 0.0)
    b = jnp.maximum(h1_ref[...], 0.0)
    o_ref[...] = (jnp.dot(a, wa_ref[...], preferred_element_type=jnp.float32)
                  + jnp.dot(b, wb_ref[...], preferred_element_type=jnp.float32))


def _combine_mm2(h0, h1, w2a, w2b):
    bm = 1000
    return pl.pallas_call(
        _mm2_body,
        grid=(N_NODES // bm,),
        in_specs=[
            pl.BlockSpec((bm, D_HALF), lambda i: (i, 0)),
            pl.BlockSpec((bm, D_HALF), lambda i: (i, 0)),
            pl.BlockSpec((D_HALF, D_OUT), lambda i: (0, 0)),
            pl.BlockSpec((D_HALF, D_OUT), lambda i: (0, 0)),
        ],
        out_specs=pl.BlockSpec((bm, D_OUT), lambda i: (i, 0)),
        out_shape=jax.ShapeDtypeStruct((N_NODES, D_OUT), jnp.float32),
    )(h0, h1, w2a, w2b)


def _add_body(a_ref, b_ref, o_ref):
    o_ref[...] = a_ref[...] + b_ref[...]


def _final_add(q0, q1):
    bm = 2000
    return pl.pallas_call(
        _add_body,
        grid=(N_NODES // bm,),
        in_specs=[
            pl.BlockSpec((bm, D_OUT), lambda i: (i, 0)),
            pl.BlockSpec((bm, D_OUT), lambda i: (i, 0)),
        ],
        out_specs=pl.BlockSpec((bm, D_OUT), lambda i: (i, 0)),
        out_shape=jax.ShapeDtypeStruct((N_NODES, D_OUT), jnp.float32),
    )(q0, q1)


def _scale_rows(rows, wv, c, d, n_edges):
    for q in range(n_edges // 16):
        wvec = wv[c, pl.ds(q * 16, 16)]
        for j in range(16):
            e = q * 16 + j
            ws = wvec[j]
            for g in range(d // 16):
                sl = pl.ds(g * 16, 16)
                rows[e, sl] = rows[e, sl] * ws


_MESH = plsc.VectorSubcoreMesh(core_axis_name="c", subcore_axis_name="s")


@functools.partial(
    pl.kernel,
    out_type=jax.ShapeDtypeStruct((2, N_PAD, D_HALF), jnp.float32),
    mesh=_MESH,
    compiler_params=_SC_PARAMS,
    scratch_types=[
        pltpu.VMEM((NCH, CH), jnp.int32),
        pltpu.VMEM((NCH, CH), jnp.int32),
        pltpu.VMEM((NCH, CH), jnp.float32),
        pltpu.VMEM((CH, D_HALF), jnp.float32),
        pltpu.VMEM_SHARED((N_PAD, D_HALF), jnp.float32),
        pltpu.SemaphoreType.DMA,
    ],
)
def _spmm1(pre_a, pre_b, src_hbm, dst_hbm, w_hbm, zero_hbm, out_hbm,
           srcv, dstv, wv, rows, acc, sem):
    cid = lax.axis_index("c")
    sid = lax.axis_index("s")

    pltpu.sync_copy(zero_hbm.at[pl.ds(sid * RPT, RPT)],
                    acc.at[pl.ds(sid * RPT, RPT)])
    pltpu.sync_copy(src_hbm.at[sid], srcv)
    pltpu.sync_copy(dst_hbm.at[sid], dstv)
    pltpu.sync_copy(w_hbm.at[sid], wv)
    plsc.subcore_barrier()

    def chunk(c, carry):
        idx = srcv.at[c]

        @pl.when(cid == 0)
        def _():
            pltpu.async_copy(pre_a.at[idx], rows, sem).wait()

        @pl.when(cid == 1)
        def _():
            pltpu.async_copy(pre_b.at[idx], rows, sem).wait()

        _scale_rows(rows, wv, c, D_HALF, CH)
        pltpu.sync_copy(rows, acc.at[dstv.at[c]], add=True)
        return carry

    lax.fori_loop(0, NCH, chunk, 0)
    plsc.subcore_barrier()

    pltpu.sync_copy(acc.at[pl.ds(sid * RPT, RPT)],
                    out_hbm.at[cid, pl.ds(sid * RPT, RPT)])


@functools.partial(
    pl.kernel,
    out_type=jax.ShapeDtypeStruct((2, N_PAD, D_OUT), jnp.float32),
    mesh=_MESH,
    compiler_params=_SC_PARAMS,
    scratch_types=[
        pltpu.VMEM((NCH // 2, CH), jnp.int32),
        pltpu.VMEM((NCH // 2, CH), jnp.int32),
        pltpu.VMEM((NCH // 2, CH), jnp.float32),
        pltpu.VMEM((CH, D_OUT), jnp.float32),
        pltpu.VMEM_SHARED((N_PAD, D_OUT), jnp.float32),
        pltpu.SemaphoreType.DMA,
    ],
)
def _spmm2(pre_hbm, src_hbm, dst_hbm, w_hbm, zero_hbm, out_hbm,
           srcv, dstv, wv, rows, acc, sem):
    cid = lax.axis_index("c")
    sid = lax.axis_index("s")
    nch = NCH // 2

    pltpu.sync_copy(zero_hbm.at[pl.ds(sid * RPT, RPT)],
                    acc.at[pl.ds(sid * RPT, RPT)])
    pltpu.sync_copy(src_hbm.at[sid, pl.ds(cid * nch, nch)], srcv)
    pltpu.sync_copy(dst_hbm.at[sid, pl.ds(cid * nch, nch)], dstv)
    pltpu.sync_copy(w_hbm.at[sid, pl.ds(cid * nch, nch)], wv)
    plsc.subcore_barrier()

    def chunk(c, carry):
        pltpu.async_copy(pre_hbm.at[srcv.at[c]], rows, sem).wait()
        _scale_rows(rows, wv, c, D_OUT, CH)
        pltpu.sync_copy(rows, acc.at[dstv.at[c]], add=True)
        return carry

    lax.fori_loop(0, nch, chunk, 0)
    plsc.subcore_barrier()

    pltpu.sync_copy(acc.at[pl.ds(sid * RPT, RPT)],
                    out_hbm.at[cid, pl.ds(sid * RPT, RPT)])


def kernel(x, edge_index, edge_weight, W1, W2):
    src = edge_index[0].astype(jnp.int32).reshape(NTILE, NCH, CH)
    dst = edge_index[1].astype(jnp.int32).reshape(NTILE, NCH, CH)
    ew = edge_weight.astype(jnp.float32).reshape(NTILE, NCH, CH)
    zero64 = jnp.zeros((N_PAD, D_HALF), jnp.float32)
    zero16 = jnp.zeros((N_PAD, D_OUT), jnp.float32)

    pre1a, pre1b = _matmul1(x, W1)
    h = _spmm1(pre1a, pre1b, src, dst, ew, zero64)
    pre2 = _combine_mm2(h[0], h[1], W2[:D_HALF], W2[D_HALF:])
    parts2 = _spmm2(pre2, src, dst, ew, zero16)
    return _final_add(parts2[0], parts2[1])


# speedup vs baseline: 6.5512x; 1.0029x over previous
import functools

import jax
import jax.numpy as jnp
from jax import lax
from jax.experimental import pallas as pl
from jax.experimental.pallas import tpu as pltpu
from jax.experimental.pallas import tpu_sc as plsc

N_NODES = 10000
N_PAD = 10240
N_EDGES = 320000
D_FEAT = 128
D_HALF = 64
D_OUT = 16

NTILE = 16
E_PAD = 327680
CH1 = 64
NCH1 = 320
CH2 = 128
NCH2 = 80
RPT = N_PAD // NTILE

_SC_PARAMS = pltpu.CompilerParams(use_tc_tiling_on_sc=False)


def _mm1_body(x_ref, w_ref, o_ref):
    res = jnp.dot(x_ref[...], w_ref[...], preferred_element_type=jnp.float32)
    o_ref[0] = res[:, :D_HALF]
    o_ref[1] = res[:, D_HALF:]


def _matmul1(x, w1):
    bm = 1000
    return pl.pallas_call(
        _mm1_body,
        grid=(N_NODES // bm,),
        in_specs=[
            pl.BlockSpec((bm, D_FEAT), lambda i: (i, 0)),
            pl.BlockSpec((D_FEAT, D_FEAT), lambda i: (0, 0)),
        ],
        out_specs=pl.BlockSpec((2, bm, D_HALF), lambda i: (0, i, 0)),
        out_shape=jax.ShapeDtypeStruct((2, N_NODES, D_HALF), jnp.float32),
    )(x, w1)


def _mm2_body(h0_ref, h1_ref, wa_ref, wb_ref, o_ref):
    a = jnp.maximum(h0_ref[...], 0.0)
    b = jnp.maximum(h1_ref[...], 0.0)
    o_ref[...] = (jnp.dot(a, wa_ref[...], preferred_element_type=jnp.float32)
                  + jnp.dot(b, wb_ref[...], preferred_element_type=jnp.float32))


def _combine_mm2(h0, h1, w2a, w2b):
    bm = 1000
    return pl.pallas_call(
        _mm2_body,
        grid=(N_NODES // bm,),
        in_specs=[
            pl.BlockSpec((bm, D_HALF), lambda i: (i, 0)),
            pl.BlockSpec((bm, D_HALF), lambda i: (i, 0)),
            pl.BlockSpec((D_HALF, D_OUT), lambda i: (0, 0)),
            pl.BlockSpec((D_HALF, D_OUT), lambda i: (0, 0)),
        ],
        out_specs=pl.BlockSpec((bm, D_OUT), lambda i: (i, 0)),
        out_shape=jax.ShapeDtypeStruct((N_NODES, D_OUT), jnp.float32),
    )(h0, h1, w2a, w2b)


def _add_body(a_ref, b_ref, o_ref):
    o_ref[...] = a_ref[...] + b_ref[...]


def _final_add(q0, q1):
    bm = 2000
    return pl.pallas_call(
        _add_body,
        grid=(N_NODES // bm,),
        in_specs=[
            pl.BlockSpec((bm, D_OUT), lambda i: (i, 0)),
            pl.BlockSpec((bm, D_OUT), lambda i: (i, 0)),
        ],
        out_specs=pl.BlockSpec((bm, D_OUT), lambda i: (i, 0)),
        out_shape=jax.ShapeDtypeStruct((N_NODES, D_OUT), jnp.float32),
    )(q0, q1)


def _scale_rows(dst, src, wv, c, d, n_edges):
    for q in range(n_edges // 16):
        wvec = wv[c, pl.ds(q * 16, 16)]
        for j in range(16):
            e = q * 16 + j
            ws = wvec[j]
            for g in range(d // 16):
                sl = pl.ds(g * 16, 16)
                dst[e, sl] = src[e, sl] * ws


_MESH = plsc.VectorSubcoreMesh(core_axis_name="c", subcore_axis_name="s")


def _make_spmm(d, nch, ch):
    npair = nch // 2

    @functools.partial(
        pl.kernel,
        out_type=jax.ShapeDtypeStruct((2, N_PAD, d), jnp.float32),
        mesh=_MESH,
        compiler_params=_SC_PARAMS,
        scratch_types=[
            pltpu.VMEM((nch, ch), jnp.int32),
            pltpu.VMEM((nch, ch), jnp.int32),
            pltpu.VMEM((nch, ch), jnp.float32),
            pltpu.VMEM((ch, d), jnp.float32),
            pltpu.VMEM((ch, d), jnp.float32),
            pltpu.VMEM((ch, d), jnp.float32),
            pltpu.VMEM((ch, d), jnp.float32),
            pltpu.VMEM_SHARED((N_PAD, d), jnp.float32),
            pltpu.SemaphoreType.DMA,
            pltpu.SemaphoreType.DMA,
            pltpu.SemaphoreType.DMA,
            pltpu.SemaphoreType.DMA,
        ],
    )
    def spmm(pre_hbm, src_hbm, dst_hbm, w_hbm, zero_hbm, out_hbm,
             srcv, dstv, wv, g0, g1, s0, s1, acc,
             gsem0, gsem1, ssem0, ssem1):
        cid = lax.axis_index("c")
        sid = lax.axis_index("s")

        pltpu.sync_copy(zero_hbm.at[pl.ds(sid * RPT, RPT)],
                        acc.at[pl.ds(sid * RPT, RPT)])
        pltpu.sync_copy(src_hbm.at[cid, sid], srcv)
        pltpu.sync_copy(dst_hbm.at[cid, sid], dstv)
        pltpu.sync_copy(w_hbm.at[cid, sid], wv)
        plsc.subcore_barrier()

        pltpu.async_copy(pre_hbm.at[srcv.at[0]], g0, gsem0)
        pltpu.async_copy(pre_hbm.at[srcv.at[1]], g1, gsem1)

        def half(i, c, gbuf, sbuf, gsem, ssem):
            pltpu.make_async_copy(pre_hbm.at[srcv.at[c]], gbuf, gsem).wait()

            @pl.when(i > 0)
            def _():
                pltpu.make_async_copy(
                    sbuf, acc.at[dstv.at[c - 2]], ssem).wait()

            _scale_rows(sbuf, gbuf, wv, c, d, ch)
            pltpu.async_copy(sbuf, acc.at[dstv.at[c]], ssem, add=True)

            @pl.when(i < npair - 1)
            def _():
                pltpu.async_copy(pre_hbm.at[srcv.at[c + 2]], gbuf, gsem)

        def pair(i, carry):
            half(i, 2 * i, g0, s0, gsem0, ssem0)
            half(i, 2 * i + 1, g1, s1, gsem1, ssem1)
            return carry

        lax.fori_loop(0, npair, pair, 0)
        pltpu.make_async_copy(s0, acc.at[dstv.at[nch - 2]], ssem0).wait()
        pltpu.make_async_copy(s1, acc.at[dstv.at[nch - 1]], ssem1).wait()
        plsc.subcore_barrier()

        pltpu.sync_copy(acc.at[pl.ds(sid * RPT, RPT)],
                        out_hbm.at[cid, pl.ds(sid * RPT, RPT)])

    return spmm


_spmm1 = _make_spmm(D_HALF, NCH1, CH1)
_spmm2 = _make_spmm(D_OUT, NCH2, CH2)


def kernel(x, edge_index, edge_weight, W1, W2):
    src = edge_index[0].astype(jnp.int32)
    dst = edge_index[1].astype(jnp.int32)
    ew = edge_weight.astype(jnp.float32)

    pad = E_PAD - N_EDGES
    src = jnp.concatenate([src, jnp.zeros((pad,), jnp.int32)])
    dst = jnp.concatenate([dst, jnp.zeros((pad,), jnp.int32)])
    ew = jnp.concatenate([ew, jnp.zeros((pad,), jnp.float32)])

    src1 = jnp.stack([src, src + N_NODES]).reshape(2, NTILE, NCH1, CH1)
    dst1 = jnp.broadcast_to(dst.reshape(1, NTILE, NCH1, CH1),
                            (2, NTILE, NCH1, CH1))
    ew1 = jnp.broadcast_to(ew.reshape(1, NTILE, NCH1, CH1),
                           (2, NTILE, NCH1, CH1))
    src2 = src.reshape(2, NTILE, NCH2, CH2)
    dst2 = dst.reshape(2, NTILE, NCH2, CH2)
    ew2 = ew.reshape(2, NTILE, NCH2, CH2)

    zero64 = jnp.zeros((N_PAD, D_HALF), jnp.float32)
    zero16 = jnp.zeros((N_PAD, D_OUT), jnp.float32)

    pre1 = _matmul1(x, W1).reshape(2 * N_NODES, D_HALF)
    h = _spmm1(pre1, src1, dst1, ew1, zero64)
    pre2 = _combine_mm2(h[0], h[1], W2[:D_HALF], W2[D_HALF:])
    parts2 = _spmm2(pre2, src2, dst2, ew2, zero16)
    return _final_add(parts2[0], parts2[1])


# speedup vs baseline: 7.7331x; 1.1804x over previous
import functools

import jax
import jax.numpy as jnp
from jax import lax
from jax.experimental import pallas as pl
from jax.experimental.pallas import tpu as pltpu
from jax.experimental.pallas import tpu_sc as plsc

N_NODES = 10000
N_PAD = 10240
N_EDGES = 320000
D_FEAT = 128
D_HALF = 64
D_OUT = 16

NTILE = 16
E_PAD = 327680
CH1 = 128
NCH1 = 160
HALVES1 = 2
CH2 = 128
NCH2 = 80
RPT = N_PAD // NTILE

_SC_PARAMS = pltpu.CompilerParams(use_tc_tiling_on_sc=False)


def _mm1_body(x_ref, w_ref, o_ref):
    res = jnp.dot(x_ref[...], w_ref[...], preferred_element_type=jnp.float32)
    o_ref[0] = res[:, :D_HALF]
    o_ref[1] = res[:, D_HALF:]


def _matmul1(x, w1):
    bm = 1000
    return pl.pallas_call(
        _mm1_body,
        grid=(N_NODES // bm,),
        in_specs=[
            pl.BlockSpec((bm, D_FEAT), lambda i: (i, 0)),
            pl.BlockSpec((D_FEAT, D_FEAT), lambda i: (0, 0)),
        ],
        out_specs=pl.BlockSpec((2, bm, D_HALF), lambda i: (0, i, 0)),
        out_shape=jax.ShapeDtypeStruct((2, N_NODES, D_HALF), jnp.float32),
    )(x, w1)


def _mm2_body(h0_ref, h1_ref, wa_ref, wb_ref, o_ref):
    a = jnp.maximum(h0_ref[...], 0.0)
    b = jnp.maximum(h1_ref[...], 0.0)
    o_ref[...] = (jnp.dot(a, wa_ref[...], preferred_element_type=jnp.float32)
                  + jnp.dot(b, wb_ref[...], preferred_element_type=jnp.float32))


def _combine_mm2(h0, h1, w2a, w2b):
    bm = 1000
    return pl.pallas_call(
        _mm2_body,
        grid=(N_NODES // bm,),
        in_specs=[
            pl.BlockSpec((bm, D_HALF), lambda i: (i, 0)),
            pl.BlockSpec((bm, D_HALF), lambda i: (i, 0)),
            pl.BlockSpec((D_HALF, D_OUT), lambda i: (0, 0)),
            pl.BlockSpec((D_HALF, D_OUT), lambda i: (0, 0)),
        ],
        out_specs=pl.BlockSpec((bm, D_OUT), lambda i: (i, 0)),
        out_shape=jax.ShapeDtypeStruct((N_NODES, D_OUT), jnp.float32),
    )(h0, h1, w2a, w2b)


def _add_body(a_ref, b_ref, o_ref):
    o_ref[...] = a_ref[...] + b_ref[...]


def _final_add(q0, q1):
    bm = 2000
    return pl.pallas_call(
        _add_body,
        grid=(N_NODES // bm,),
        in_specs=[
            pl.BlockSpec((bm, D_OUT), lambda i: (i, 0)),
            pl.BlockSpec((bm, D_OUT), lambda i: (i, 0)),
        ],
        out_specs=pl.BlockSpec((bm, D_OUT), lambda i: (i, 0)),
        out_shape=jax.ShapeDtypeStruct((N_NODES, D_OUT), jnp.float32),
    )(q0, q1)


def _scale_rows(dst, src, wv, c, d, n_edges):
    for q in range(n_edges // 16):
        wvec = wv[c, pl.ds(q * 16, 16)]
        for j in range(16):
            e = q * 16 + j
            ws = wvec[j]
            for g in range(d // 16):
                sl = pl.ds(g * 16, 16)
                dst[e, sl] = src[e, sl] * ws


_MESH = plsc.VectorSubcoreMesh(core_axis_name="c", subcore_axis_name="s")


def _make_spmm(d, nch, ch, halves=1):
    nbuf = nch // halves
    npair = nbuf // 2

    @functools.partial(
        pl.kernel,
        out_type=jax.ShapeDtypeStruct((2, N_PAD, d), jnp.float32),
        mesh=_MESH,
        compiler_params=_SC_PARAMS,
        scratch_types=[
            pltpu.VMEM((nbuf, ch), jnp.int32),
            pltpu.VMEM((nbuf, ch), jnp.int32),
            pltpu.VMEM((nbuf, ch), jnp.float32),
            pltpu.VMEM((ch, d), jnp.float32),
            pltpu.VMEM((ch, d), jnp.float32),
            pltpu.VMEM((ch, d), jnp.float32),
            pltpu.VMEM((ch, d), jnp.float32),
            pltpu.VMEM_SHARED((N_PAD, d), jnp.float32),
            pltpu.SemaphoreType.DMA,
            pltpu.SemaphoreType.DMA,
            pltpu.SemaphoreType.DMA,
            pltpu.SemaphoreType.DMA,
        ],
    )
    def spmm(pre_hbm, src_hbm, dst_hbm, w_hbm, zero_hbm, out_hbm,
             srcv, dstv, wv, g0, g1, s0, s1, acc,
             gsem0, gsem1, ssem0, ssem1):
        cid = lax.axis_index("c")
        sid = lax.axis_index("s")

        pltpu.sync_copy(zero_hbm.at[pl.ds(sid * RPT, RPT)],
                        acc.at[pl.ds(sid * RPT, RPT)])
        plsc.subcore_barrier()

        def half(i, c, gbuf, sbuf, gsem, ssem):
            pltpu.make_async_copy(pre_hbm.at[srcv.at[c]], gbuf, gsem).wait()

            @pl.when(i > 0)
            def _():
                pltpu.make_async_copy(
                    sbuf, acc.at[dstv.at[c - 2]], ssem).wait()

            _scale_rows(sbuf, gbuf, wv, c, d, ch)
            pltpu.async_copy(sbuf, acc.at[dstv.at[c]], ssem, add=True)

            @pl.when(i < npair - 1)
            def _():
                pltpu.async_copy(pre_hbm.at[srcv.at[c + 2]], gbuf, gsem)

        def pair(i, carry):
            half(i, 2 * i, g0, s0, gsem0, ssem0)
            half(i, 2 * i + 1, g1, s1, gsem1, ssem1)
            return carry

        def stage(hv, carry):
            pltpu.sync_copy(src_hbm.at[cid, sid, pl.ds(hv * nbuf, nbuf)],
                            srcv)
            pltpu.sync_copy(dst_hbm.at[cid, sid, pl.ds(hv * nbuf, nbuf)],
                            dstv)
            pltpu.sync_copy(w_hbm.at[cid, sid, pl.ds(hv * nbuf, nbuf)], wv)
            pltpu.async_copy(pre_hbm.at[srcv.at[0]], g0, gsem0)
            pltpu.async_copy(pre_hbm.at[srcv.at[1]], g1, gsem1)
            lax.fori_loop(0, npair, pair, 0)
            pltpu.make_async_copy(s0, acc.at[dstv.at[nbuf - 2]], ssem0).wait()
            pltpu.make_async_copy(s1, acc.at[dstv.at[nbuf - 1]], ssem1).wait()
            return carry

        lax.fori_loop(0, halves, stage, 0)
        plsc.subcore_barrier()

        pltpu.sync_copy(acc.at[pl.ds(sid * RPT, RPT)],
                        out_hbm.at[cid, pl.ds(sid * RPT, RPT)])

    return spmm


_spmm1 = _make_spmm(D_HALF, NCH1, CH1, HALVES1)
_spmm2 = _make_spmm(D_OUT, NCH2, CH2)


def kernel(x, edge_index, edge_weight, W1, W2):
    src = edge_index[0].astype(jnp.int32)
    dst = edge_index[1].astype(jnp.int32)
    ew = edge_weight.astype(jnp.float32)

    pad = E_PAD - N_EDGES
    src = jnp.concatenate([src, jnp.zeros((pad,), jnp.int32)])
    dst = jnp.concatenate([dst, jnp.zeros((pad,), jnp.int32)])
    ew = jnp.concatenate([ew, jnp.zeros((pad,), jnp.float32)])

    src1 = jnp.stack([src, src + N_NODES]).reshape(2, NTILE, NCH1, CH1)
    dst1 = jnp.broadcast_to(dst.reshape(1, NTILE, NCH1, CH1),
                            (2, NTILE, NCH1, CH1))
    ew1 = jnp.broadcast_to(ew.reshape(1, NTILE, NCH1, CH1),
                           (2, NTILE, NCH1, CH1))
    src2 = src.reshape(2, NTILE, NCH2, CH2)
    dst2 = dst.reshape(2, NTILE, NCH2, CH2)
    ew2 = ew.reshape(2, NTILE, NCH2, CH2)

    zero64 = jnp.zeros((N_PAD, D_HALF), jnp.float32)
    zero16 = jnp.zeros((N_PAD, D_OUT), jnp.float32)

    pre1 = _matmul1(x, W1).reshape(2 * N_NODES, D_HALF)
    h = _spmm1(pre1, src1, dst1, ew1, zero64)
    pre2 = _combine_mm2(h[0], h[1], W2[:D_HALF], W2[D_HALF:])
    parts2 = _spmm2(pre2, src2, dst2, ew2, zero16)
    return _final_add(parts2[0], parts2[1])
